# Initial kernel scaffold; baseline (speedup 1.0000x reference)
import jax
import jax.numpy as jnp
from jax.experimental import pallas as pl


def kernel(x, tables, W, b, gamma, beta):
    raise NotImplementedError("write your pallas kernel here")


# speedup vs baseline: 8.0193x; 8.0193x over previous
import functools

import jax
import jax.numpy as jnp
from jax import lax
from jax.experimental import pallas as pl
from jax.experimental.pallas import tpu as pltpu
from jax.experimental.pallas import tpu_sc as plsc

F = 26
V = 100000
E = 32
OUT = 128
B = 16384
EPS = 1e-5

NW = 32
ROWS = B * F
IDX_MINOR = 128
TILE_ROWS = 8
CHUNK = TILE_ROWS * IDX_MINOR
PER_W_TILES = ROWS // IDX_MINOR // NW
STEPS = PER_W_TILES // TILE_ROWS


def _sc_gather(table_flat, idx2):
    mesh = plsc.VectorSubcoreMesh(core_axis_name="c", subcore_axis_name="s")

    @functools.partial(
        pl.kernel,
        mesh=mesh,
        out_type=jax.ShapeDtypeStruct((ROWS, E), jnp.float32),
        scratch_types=[
            pltpu.VMEM((TILE_ROWS, IDX_MINOR), jnp.int32),
            pltpu.VMEM((CHUNK, E), jnp.float32),
            pltpu.SemaphoreType.DMA,
        ],
        compiler_params=pltpu.CompilerParams(use_tc_tiling_on_sc=False),
    )
    def k(table_hbm, idx_hbm, out_hbm, idx_v, rows_v, sem):
        wid = lax.axis_index("s") * 2 + lax.axis_index("c")
        tile_base = wid * PER_W_TILES

        def body(i, carry):
            t0 = tile_base + i * TILE_ROWS
            pltpu.sync_copy(idx_hbm.at[pl.ds(t0, TILE_ROWS)], idx_v)
            cps = [
                pltpu.async_copy(
                    table_hbm.at[idx_v.at[j]],
                    rows_v.at[pl.ds(j * IDX_MINOR, IDX_MINOR)],
                    sem,
                )
                for j in range(TILE_ROWS)
            ]
            for cp in cps:
                cp.wait()
            pltpu.sync_copy(rows_v, out_hbm.at[pl.ds(t0 * IDX_MINOR, CHUNK)])
            return carry

        lax.fori_loop(0, STEPS, body, 0)

    return k(table_flat, idx2)


def _tc_proj(emb, W, b, gamma, beta):
    BB = 512

    def body(e_ref, w_ref, b_ref, g_ref, bt_ref, o_ref):
        h = jnp.dot(e_ref[...], w_ref[...], preferred_element_type=jnp.float32)
        h = jnp.maximum(h + b_ref[...], 0.0)
        m = jnp.mean(h, axis=-1, keepdims=True)
        c = h - m
        v = jnp.mean(c * c, axis=-1, keepdims=True)
        o_ref[...] = c * lax.rsqrt(v + EPS) * g_ref[...] + bt_ref[...]

    return pl.pallas_call(
        body,
        grid=(B // BB,),
        in_specs=[
            pl.BlockSpec((BB, F * E), lambda i: (i, 0)),
            pl.BlockSpec((F * E, OUT), lambda i: (0, 0)),
            pl.BlockSpec((1, OUT), lambda i: (0, 0)),
            pl.BlockSpec((1, OUT), lambda i: (0, 0)),
            pl.BlockSpec((1, OUT), lambda i: (0, 0)),
        ],
        out_specs=pl.BlockSpec((BB, OUT), lambda i: (i, 0)),
        out_shape=jax.ShapeDtypeStruct((B, OUT), jnp.float32),
    )(emb, W, b.reshape(1, OUT), gamma.reshape(1, OUT), beta.reshape(1, OUT))


def kernel(x, tables, W, b, gamma, beta):
    offs = (jnp.arange(F, dtype=jnp.int32) * V)[None, :]
    idx = (x.astype(jnp.int32) + offs).reshape(ROWS // IDX_MINOR, IDX_MINOR)
    table_flat = tables.reshape(F * V, E)
    emb = _sc_gather(table_flat, idx)
    return _tc_proj(emb.reshape(B, F * E), W, b, gamma, beta)
